# baseline (speedup 1.0000x reference)
import functools

import jax
import jax.numpy as jnp
from jax import lax
from jax.experimental import pallas as pl
from jax.experimental.pallas import tpu as pltpu
from jax.experimental.pallas import tpu_sc as plsc

RES_LAMBDA = 0.5
N_HOPS = 2

N = 10000
E = 160000
C = 256
H = C // 2
R = 64
NC = 2
NS = 16
NW = NC * NS
EW2 = E // NW
CH = 128
NCH2 = (EW2 + CH - 1) // CH
EW2_PAD = NCH2 * CH
EW4 = E // NS
CH4 = 96
NCH4 = 108
EW4_PAD = NCH4 * CH4
N2 = 10240
STRIPE = N2 // NS
NP16 = N + 16
GB = 2000

_mesh = plsc.VectorSubcoreMesh(core_axis_name="c", subcore_axis_name="s")
_sc_params = pltpu.CompilerParams(needs_layout_passes=False)


def _pq_body(cur_ref, rel_ref, w_ref, pq_ref):
    V = lax.dot_general(w_ref[...], rel_ref[...], (((1,), (1,)), ((), ())),
                        preferred_element_type=jnp.float32)
    Vcat = jnp.concatenate([V[:C], V[C:]], axis=1)
    pq_ref[...] = jnp.dot(cur_ref[...], Vcat, preferred_element_type=jnp.float32)


_pq_call = pl.pallas_call(
    _pq_body,
    out_shape=jax.ShapeDtypeStruct((N, 2 * R), jnp.float32),
)


def _mhat_body(s1p_ref, mhat_ref):
    s = jnp.sum(s1p_ref[...], axis=0, keepdims=True)
    mhat_ref[...] = 4.0 * jnp.log(s + 1e-37)


_mhat_call = pl.pallas_call(
    _mhat_body,
    out_shape=jax.ShapeDtypeStruct((1, N), jnp.float32),
)


@functools.partial(
    pl.kernel,
    out_type=(
        jax.ShapeDtypeStruct((E,), jnp.float32),
        jax.ShapeDtypeStruct((NW * N,), jnp.float32),
    ),
    mesh=_mesh,
    compiler_params=_sc_params,
    scratch_types=[
        pltpu.VMEM((NCH2, CH), jnp.int32),
        pltpu.VMEM((NCH2, CH), jnp.int32),
        pltpu.VMEM((NCH2, CH), jnp.int32),
        pltpu.VMEM((EW2_PAD,), jnp.float32),
        pltpu.VMEM((NP16,), jnp.float32),
        pltpu.VMEM((CH,), jnp.float32),
        pltpu.VMEM((CH,), jnp.float32),
        pltpu.SemaphoreType.DMA,
        pltpu.SemaphoreType.DMA,
    ],
)
def _edge_pass1(pq_hbm, idxp_hbm, idxq_hbm, hpad_hbm, e_out, s1_out,
                idxp_v, idxq_v, h_v, e_v, s1_v, gp_v, gq_v, sem_p, sem_q):
    cid = lax.axis_index("c")
    sid = lax.axis_index("s")
    w = sid * NC + cid

    pltpu.sync_copy(idxp_hbm.at[w], idxp_v)
    pltpu.sync_copy(idxq_hbm.at[w], idxq_v)
    pltpu.sync_copy(hpad_hbm.at[w], h_v)

    @pl.loop(0, NP16 // 16)
    def _zero(i):
        s1_v[pl.ds(i * 16, 16)] = jnp.zeros((16,), jnp.float32)

    @pl.loop(0, NCH2)
    def _chunk(ci):
        cp = pltpu.async_copy(pq_hbm.at[idxp_v.at[ci]], gp_v, sem_p)
        cq = pltpu.async_copy(pq_hbm.at[idxq_v.at[ci]], gq_v, sem_q)
        cp.wait()
        cq.wait()
        cbase = ci * CH
        for j in range(CH // 16):
            sl = pl.ds(j * 16, 16)
            x = gp_v[sl] + gq_v[sl]
            e16 = jnp.where(x > 0, x, 0.2 * x)
            e_v[pl.ds(cbase + j * 16, 16)] = e16
            z = jnp.exp(e16 * 0.25)
            plsc.addupdate_scatter(s1_v, [h_v[ci, sl]], z)

    pltpu.sync_copy(e_v.at[pl.ds(0, EW2)], e_out.at[pl.ds(w * EW2, EW2)])
    pltpu.sync_copy(s1_v.at[pl.ds(0, N)], s1_out.at[pl.ds(w * N, N)])


@functools.partial(
    pl.kernel,
    out_type=(
        jax.ShapeDtypeStruct((E,), jnp.float32),
        jax.ShapeDtypeStruct((NW * N,), jnp.float32),
    ),
    mesh=_mesh,
    compiler_params=_sc_params,
    scratch_types=[
        pltpu.VMEM((NCH2, CH), jnp.int32),
        pltpu.VMEM((EW2_PAD,), jnp.float32),
        pltpu.VMEM((EW2_PAD,), jnp.float32),
        pltpu.VMEM((NP16,), jnp.float32),
        pltpu.VMEM((CH,), jnp.float32),
        pltpu.SemaphoreType.DMA,
    ],
)
def _edge_weights(e_hbm, mhat_hbm, hpad_hbm, u_out, s2_out,
                  h_v, e_v, u_v, s2_v, m_v, sem_m):
    cid = lax.axis_index("c")
    sid = lax.axis_index("s")
    w = sid * NC + cid

    pltpu.sync_copy(hpad_hbm.at[w], h_v)
    pltpu.sync_copy(e_hbm.at[pl.ds(w * EW2, EW2)], e_v.at[pl.ds(0, EW2)])

    @pl.loop(0, (EW2_PAD - EW2) // 16)
    def _zero_tail(i):
        e_v[pl.ds(EW2 + i * 16, 16)] = jnp.zeros((16,), jnp.float32)

    @pl.loop(0, NP16 // 16)
    def _zero(i):
        s2_v[pl.ds(i * 16, 16)] = jnp.zeros((16,), jnp.float32)

    @pl.loop(0, NCH2)
    def _chunk(ci):
        pltpu.async_copy(mhat_hbm.at[h_v.at[ci]], m_v, sem_m).wait()
        cbase = ci * CH
        for j in range(CH // 16):
            sl = pl.ds(j * 16, 16)
            u16 = jnp.exp(e_v[pl.ds(cbase + j * 16, 16)] - m_v[sl])
            u_v[pl.ds(cbase + j * 16, 16)] = u16
            plsc.addupdate_scatter(s2_v, [h_v[ci, sl]], u16)

    pltpu.sync_copy(u_v.at[pl.ds(0, EW2)], u_out.at[pl.ds(w * EW2, EW2)])
    pltpu.sync_copy(s2_v.at[pl.ds(0, N)], s2_out.at[pl.ds(w * N, N)])


@functools.partial(
    pl.kernel,
    out_type=jax.ShapeDtypeStruct((NC * N2, H), jnp.float32),
    mesh=_mesh,
    compiler_params=_sc_params,
    scratch_types=[
        pltpu.VMEM((3, CH4), jnp.int32),
        pltpu.VMEM((3, CH4), jnp.int32),
        pltpu.VMEM((3, CH4), jnp.int32),
        pltpu.VMEM((CH4,), jnp.int32),
        pltpu.VMEM((CH4,), jnp.int32),
        pltpu.VMEM((CH4,), jnp.int32),
        pltpu.VMEM((CH4,), jnp.float32),
        pltpu.VMEM((CH4,), jnp.float32),
        pltpu.VMEM((CH4,), jnp.float32),
        pltpu.VMEM((CH4, H), jnp.float32),
        pltpu.VMEM((CH4, H), jnp.float32),
        pltpu.VMEM((CH4, H), jnp.float32),
        pltpu.VMEM_SHARED((N2, H), jnp.float32),
        pltpu.SemaphoreType.DMA,
        pltpu.SemaphoreType.DMA,
        pltpu.SemaphoreType.DMA,
        pltpu.SemaphoreType.DMA,
        pltpu.SemaphoreType.DMA,
        pltpu.SemaphoreType.DMA,
    ],
)
def _edge_pass2(emb2_hbm, het_hbm,
                u_out,
                het0, het1, het2, tidx0, tidx1, tidx2, u0, u1, u2,
                rows0, rows1, rows2, U_sh,
                sg0, sg1, sg2, ss0, ss1, ss2):
    cid = lax.axis_index("c")
    sid = lax.axis_index("s")
    HET = [het0, het1, het2]
    TIDX = [tidx0, tidx1, tidx2]
    UV = [u0, u1, u2]
    ROWS = [rows0, rows1, rows2]
    SG = [sg0, sg1, sg2]
    SS = [ss0, ss1, ss2]

    @pl.loop(0, CH4)
    def _zero_rows(i):
        for cb in range(H // 16):
            z16 = jnp.zeros((16,), jnp.float32)
            rows0[i, pl.ds(cb * 16, 16)] = z16

    o = 0
    for n in [CH4] * (STRIPE // CH4) + [STRIPE - (STRIPE // CH4) * CH4]:
        if n:
            pltpu.sync_copy(rows0.at[pl.ds(0, n)],
                            U_sh.at[pl.ds(sid * STRIPE + o, n)])
            o += n
    plsc.subcore_barrier()

    def _load_het(ci, b):
        pltpu.sync_copy(het_hbm.at[sid, ci], HET[b])
        for j in range(CH4 // 16):
            sl = pl.ds(j * 16, 16)
            TIDX[b][sl] = HET[b][1, sl] + cid
            UV[b][sl] = plsc.bitcast(HET[b][2, sl], jnp.float32)

    def _issue_gather(b):
        pltpu.async_copy(emb2_hbm.at[TIDX[b]], ROWS[b], SG[b])

    def _wait_gather(b):
        pltpu.make_async_copy(emb2_hbm.at[TIDX[b]], ROWS[b], SG[b]).wait()

    def _issue_scatter(b):
        pltpu.async_copy(ROWS[b], U_sh.at[HET[b].at[0]], SS[b], add=True)

    def _wait_scatter(b):
        pltpu.make_async_copy(ROWS[b], U_sh.at[HET[b].at[0]], SS[b]).wait()

    _load_het(0, 0)
    _issue_gather(0)
    _load_het(1, 1)
    _issue_gather(1)

    @pl.loop(0, NCH4 // 3)
    def _ring(k):
        for i in range(3):
            b = i
            bp = (i + 2) % 3
            c = 3 * k + i
            if i == 0:
                @pl.when(k > 0)
                def _w0():
                    _wait_scatter(bp)
            else:
                _wait_scatter(bp)

            @pl.when(c + 2 < NCH4)
            def _pref():
                _load_het(c + 2, bp)
                _issue_gather(bp)

            _wait_gather(b)

            @pl.loop(0, CH4)
            def _scale(r):
                ub = plsc.load_gather(UV[b], [jnp.zeros((16,), jnp.int32) + r])
                for cb in range(H // 16):
                    sl = pl.ds(cb * 16, 16)
                    ROWS[b][r, sl] = ROWS[b][r, sl] * ub

            _issue_scatter(b)

    _wait_scatter((NCH4 - 1) % 3)
    plsc.subcore_barrier()
    pltpu.sync_copy(U_sh.at[pl.ds(sid * STRIPE, STRIPE)],
                    u_out.at[pl.ds(cid * N2 + sid * STRIPE, STRIPE)])


def _finish_body(u0_ref, u1_ref, s2t_ref, cur_ref, res_ref, cur_out, res_out):
    U = jnp.concatenate([u0_ref[...], u1_ref[...]], axis=1)
    s2 = jnp.sum(s2t_ref[...], axis=1, keepdims=True)
    agg = U / (s2 + 1e-16)
    cur2 = agg + cur_ref[...]
    n2 = jnp.sum(cur2 * cur2, axis=1, keepdims=True)
    curn = cur2 / jnp.maximum(jnp.sqrt(n2), 1e-12)
    cur_out[...] = curn
    res_out[...] = RES_LAMBDA * res_ref[...] + curn


_finish_call = pl.pallas_call(
    _finish_body,
    grid=(N // GB,),
    in_specs=[
        pl.BlockSpec((GB, H), lambda i: (i, 0)),
        pl.BlockSpec((GB, H), lambda i: (i, 0)),
        pl.BlockSpec((GB, NW), lambda i: (i, 0)),
        pl.BlockSpec((GB, C), lambda i: (i, 0)),
        pl.BlockSpec((GB, C), lambda i: (i, 0)),
    ],
    out_specs=(
        pl.BlockSpec((GB, C), lambda i: (i, 0)),
        pl.BlockSpec((GB, C), lambda i: (i, 0)),
    ),
    out_shape=(
        jax.ShapeDtypeStruct((N, C), jnp.float32),
        jax.ShapeDtypeStruct((N, C), jnp.float32),
    ),
)


def kernel(entity_emb, relation_emb, edge_index, edge_type, W):
    head = edge_index[0].astype(jnp.int32)
    tail = edge_index[1].astype(jnp.int32)
    et = edge_type.astype(jnp.int32)

    pad2 = ((0, 0), (0, EW2_PAD - EW2))
    h2 = head.reshape(NW, EW2)
    y2 = et.reshape(NW, EW2)
    t2 = tail.reshape(NW, EW2)
    idxp2 = jnp.pad(h2 * (2 * R) + y2, pad2, constant_values=N * 2 * R)
    idxq2 = jnp.pad(t2 * (2 * R) + R + y2, pad2, constant_values=N * 2 * R)
    hpad2 = jnp.pad(h2, pad2, constant_values=N).reshape(NW, NCH2, CH)
    idxp2 = idxp2.reshape(NW, NCH2, CH)
    idxq2 = idxq2.reshape(NW, NCH2, CH)

    pad4 = ((0, 0), (0, EW4_PAD - EW4))
    hpad4 = jnp.pad(head.reshape(NS, EW4), pad4, constant_values=N)
    t2pad4 = jnp.pad(tail.reshape(NS, EW4) * NC, pad4, constant_values=NC * N)

    res = entity_emb
    cur = entity_emb
    for _ in range(N_HOPS):
        pq = _pq_call(cur, relation_emb, W)
        pqpad = jnp.pad(pq.reshape(-1), (0, 2 * R))
        e, s1p = _edge_pass1(pqpad, idxp2, idxq2, hpad2)
        s1p = s1p.reshape(NW, N)
        mhat = jnp.pad(_mhat_call(s1p).reshape(-1), (0, 16))
        u, s2p = _edge_weights(e, mhat, hpad2)
        s2p = s2p.reshape(NW, N)
        ubits = lax.bitcast_convert_type(
            jnp.pad(u.reshape(NS, EW4), pad4, constant_values=0.0), jnp.int32)
        het = jnp.stack([hpad4, t2pad4, ubits], axis=2)
        het = het.reshape(NS, NCH4, CH4, 3).transpose(0, 1, 3, 2)
        emb2 = jnp.pad(cur.reshape(NC * N, H), ((0, NC), (0, 0)))
        u_halves = _edge_pass2(emb2, het)
        cur, res = _finish_call(
            u_halves[:N2], u_halves[N2:], jnp.transpose(s2p), cur, res)
    return res


# speedup vs baseline: 1.3530x; 1.3530x over previous
import functools

import jax
import jax.numpy as jnp
from jax import lax
from jax.experimental import pallas as pl
from jax.experimental.pallas import tpu as pltpu
from jax.experimental.pallas import tpu_sc as plsc

RES_LAMBDA = 0.5
N_HOPS = 2

N = 10000
E = 160000
C = 256
H = C // 2
R = 64
NC = 2
NS = 16
NW = NC * NS
EW2 = E // NW
CH = 128
NCH2 = (EW2 + CH - 1) // CH
EW2_PAD = NCH2 * CH
EW4 = E // NS
CH4 = 96
NCH4 = 106
EW4_PAD = NCH4 * CH4
N2 = 10240
STRIPE = N2 // NS
NP16 = N + 16
GB = 2000

_mesh = plsc.VectorSubcoreMesh(core_axis_name="c", subcore_axis_name="s")
_sc_params = pltpu.CompilerParams(needs_layout_passes=False)


def _pq_body(cur_ref, rel_ref, w_ref, pq_ref):
    V = lax.dot_general(w_ref[...], rel_ref[...], (((1,), (1,)), ((), ())),
                        preferred_element_type=jnp.float32)
    Vcat = jnp.concatenate([V[:C], V[C:]], axis=1)
    pq_ref[...] = jnp.dot(cur_ref[...], Vcat, preferred_element_type=jnp.float32)


_pq_call = pl.pallas_call(
    _pq_body,
    out_shape=jax.ShapeDtypeStruct((N, 2 * R), jnp.float32),
)


def _mhat_body(s1p_ref, mhat_ref):
    s = jnp.sum(s1p_ref[...], axis=0, keepdims=True)
    mhat_ref[...] = 4.0 * jnp.log(s + 1e-37)


_mhat_call = pl.pallas_call(
    _mhat_body,
    out_shape=jax.ShapeDtypeStruct((1, N), jnp.float32),
)


@functools.partial(
    pl.kernel,
    out_type=(
        jax.ShapeDtypeStruct((E,), jnp.float32),
        jax.ShapeDtypeStruct((NW * N,), jnp.float32),
    ),
    mesh=_mesh,
    compiler_params=_sc_params,
    scratch_types=[
        pltpu.VMEM((NCH2, CH), jnp.int32),
        pltpu.VMEM((NCH2, CH), jnp.int32),
        pltpu.VMEM((NCH2, CH), jnp.int32),
        pltpu.VMEM((EW2_PAD,), jnp.float32),
        pltpu.VMEM((NP16,), jnp.float32),
        pltpu.VMEM((2, CH), jnp.float32),
        pltpu.VMEM((2, CH), jnp.float32),
        pltpu.SemaphoreType.DMA,
        pltpu.SemaphoreType.DMA,
        pltpu.SemaphoreType.DMA,
        pltpu.SemaphoreType.DMA,
    ],
)
def _edge_pass1(pq_hbm, idxp_hbm, idxq_hbm, hpad_hbm, e_out, s1_out,
                idxp_v, idxq_v, h_v, e_v, s1_v, gp_v, gq_v,
                sp0, sq0, sp1, sq1):
    cid = lax.axis_index("c")
    sid = lax.axis_index("s")
    w = sid * NC + cid

    pltpu.sync_copy(idxp_hbm.at[w], idxp_v)
    pltpu.sync_copy(idxq_hbm.at[w], idxq_v)
    pltpu.sync_copy(hpad_hbm.at[w], h_v)

    SP = [sp0, sp1]
    SQ = [sq0, sq1]

    @pl.loop(0, NP16 // 16)
    def _zero(i):
        s1_v[pl.ds(i * 16, 16)] = jnp.zeros((16,), jnp.float32)

    def _issue(ci, b):
        pltpu.async_copy(pq_hbm.at[idxp_v.at[ci]], gp_v.at[b], SP[b])
        pltpu.async_copy(pq_hbm.at[idxq_v.at[ci]], gq_v.at[b], SQ[b])

    def _wait(ci, b):
        pltpu.make_async_copy(pq_hbm.at[idxp_v.at[ci]], gp_v.at[b], SP[b]).wait()
        pltpu.make_async_copy(pq_hbm.at[idxq_v.at[ci]], gq_v.at[b], SQ[b]).wait()

    _issue(0, 0)
    _issue(1, 1)

    @pl.loop(0, NCH2 // 2)
    def _chunk(k):
        for i in range(2):
            ci = 2 * k + i
            _wait(ci, i)
            cbase = ci * CH
            for j in range(CH // 16):
                sl = pl.ds(j * 16, 16)
                x = gp_v[i, sl] + gq_v[i, sl]
                e16 = jnp.where(x > 0, x, 0.2 * x)
                e_v[pl.ds(cbase + j * 16, 16)] = e16
                z = jnp.exp(e16 * 0.25)
                plsc.addupdate_scatter(s1_v, [h_v[ci, sl]], z)

            @pl.when(ci + 2 < NCH2)
            def _pref():
                _issue(ci + 2, i)

    pltpu.sync_copy(e_v.at[pl.ds(0, EW2)], e_out.at[pl.ds(w * EW2, EW2)])
    pltpu.sync_copy(s1_v.at[pl.ds(0, N)], s1_out.at[pl.ds(w * N, N)])


@functools.partial(
    pl.kernel,
    out_type=(
        jax.ShapeDtypeStruct((NC * N2, H), jnp.float32),
        jax.ShapeDtypeStruct((NW * N,), jnp.float32),
    ),
    mesh=_mesh,
    compiler_params=_sc_params,
    scratch_types=[
        pltpu.VMEM((3, CH4), jnp.int32),
        pltpu.VMEM((3, CH4), jnp.int32),
        pltpu.VMEM((CH4,), jnp.int32),
        pltpu.VMEM((CH4,), jnp.int32),
        pltpu.VMEM((NP16,), jnp.float32),
        pltpu.VMEM((NP16,), jnp.float32),
        pltpu.VMEM((CH4,), jnp.float32),
        pltpu.VMEM((CH4, H), jnp.float32),
        pltpu.VMEM((CH4, H), jnp.float32),
        pltpu.VMEM_SHARED((N2, H), jnp.float32),
        pltpu.SemaphoreType.DMA,
        pltpu.SemaphoreType.DMA,
    ],
)
def _edge_pass2(emb2_hbm, het_hbm, mhat_hbm,
                u_out, s2_out,
                hetA, hetB, tidxA, tidxB, mhat_v, s2_v,
                u_v, rowsA, rowsB, U_sh, semA, semB):
    cid = lax.axis_index("c")
    sid = lax.axis_index("s")
    w = sid * NC + cid

    pltpu.sync_copy(mhat_hbm, mhat_v)

    @pl.loop(0, NP16 // 16)
    def _zero_s2(i):
        s2_v[pl.ds(i * 16, 16)] = jnp.zeros((16,), jnp.float32)

    @pl.loop(0, CH4)
    def _zero_rows(i):
        for cb in range(H // 16):
            z16 = jnp.zeros((16,), jnp.float32)
            rowsA[i, pl.ds(cb * 16, 16)] = z16
            rowsB[i, pl.ds(cb * 16, 16)] = z16

    o = 0
    for n in [CH4] * (STRIPE // CH4) + [STRIPE - (STRIPE // CH4) * CH4]:
        if n:
            pltpu.sync_copy(rowsA.at[pl.ds(0, n)],
                            U_sh.at[pl.ds(sid * STRIPE + o, n)])
            o += n
    plsc.subcore_barrier()

    def _load_het(ci, het_v, tidx_v):
        pltpu.sync_copy(het_hbm.at[sid, ci], het_v)
        for j in range(CH4 // 16):
            sl = pl.ds(j * 16, 16)
            tidx_v[sl] = het_v[1, sl] + cid

    def _issue_gather(tidx_v, rows_v, sem):
        return pltpu.async_copy(emb2_hbm.at[tidx_v], rows_v, sem)

    def _process(het_v, rows_v):
        for j in range(CH4 // 16):
            sl = pl.ds(j * 16, 16)
            h16 = het_v[0, sl]
            m16 = plsc.load_gather(mhat_v, [h16])
            e16 = plsc.bitcast(het_v[2, sl], jnp.float32)
            u16 = jnp.exp(e16 - m16)
            u_v[sl] = u16
            plsc.addupdate_scatter(s2_v, [h16], u16)

        @pl.loop(0, CH4)
        def _scale(r):
            ub = plsc.load_gather(u_v, [jnp.zeros((16,), jnp.int32) + r])
            for cb in range(H // 16):
                sl = pl.ds(cb * 16, 16)
                rows_v[r, sl] = rows_v[r, sl] * ub

        pltpu.sync_copy(rows_v, U_sh.at[het_v.at[0]], add=True)

    _load_het(0, hetA, tidxA)
    cpA0 = _issue_gather(tidxA, rowsA, semA)
    _load_het(1, hetB, tidxB)
    cpB0 = _issue_gather(tidxB, rowsB, semB)

    @pl.loop(0, NCH4 // 2)
    def _pair(k):
        cpA = pltpu.make_async_copy(emb2_hbm.at[tidxA], rowsA, semA)
        cpA.wait()
        _process(hetA, rowsA)

        @pl.when(k < NCH4 // 2 - 1)
        def _prefA():
            _load_het(2 * k + 2, hetA, tidxA)
            _issue_gather(tidxA, rowsA, semA)

        cpB = pltpu.make_async_copy(emb2_hbm.at[tidxB], rowsB, semB)
        cpB.wait()
        _process(hetB, rowsB)

        @pl.when(k < NCH4 // 2 - 1)
        def _prefB():
            _load_het(2 * k + 3, hetB, tidxB)
            _issue_gather(tidxB, rowsB, semB)

    plsc.subcore_barrier()
    pltpu.sync_copy(U_sh.at[pl.ds(sid * STRIPE, STRIPE)],
                    u_out.at[pl.ds(cid * N2 + sid * STRIPE, STRIPE)])
    pltpu.sync_copy(s2_v.at[pl.ds(0, N)], s2_out.at[pl.ds(w * N, N)])


def _finish_body(u0_ref, u1_ref, s2t_ref, cur_ref, res_ref, cur_out, res_out):
    U = jnp.concatenate([u0_ref[...], u1_ref[...]], axis=1)
    s2 = jnp.sum(s2t_ref[...], axis=1, keepdims=True) * 0.5
    agg = U / (s2 + 1e-16)
    cur2 = agg + cur_ref[...]
    n2 = jnp.sum(cur2 * cur2, axis=1, keepdims=True)
    curn = cur2 / jnp.maximum(jnp.sqrt(n2), 1e-12)
    cur_out[...] = curn
    res_out[...] = RES_LAMBDA * res_ref[...] + curn


_finish_call = pl.pallas_call(
    _finish_body,
    grid=(N // GB,),
    in_specs=[
        pl.BlockSpec((GB, H), lambda i: (i, 0)),
        pl.BlockSpec((GB, H), lambda i: (i, 0)),
        pl.BlockSpec((GB, NW), lambda i: (i, 0)),
        pl.BlockSpec((GB, C), lambda i: (i, 0)),
        pl.BlockSpec((GB, C), lambda i: (i, 0)),
    ],
    out_specs=(
        pl.BlockSpec((GB, C), lambda i: (i, 0)),
        pl.BlockSpec((GB, C), lambda i: (i, 0)),
    ),
    out_shape=(
        jax.ShapeDtypeStruct((N, C), jnp.float32),
        jax.ShapeDtypeStruct((N, C), jnp.float32),
    ),
)


def kernel(entity_emb, relation_emb, edge_index, edge_type, W):
    head = edge_index[0].astype(jnp.int32)
    tail = edge_index[1].astype(jnp.int32)
    et = edge_type.astype(jnp.int32)

    pad2 = ((0, 0), (0, EW2_PAD - EW2))
    h2 = head.reshape(NW, EW2)
    y2 = et.reshape(NW, EW2)
    t2 = tail.reshape(NW, EW2)
    idxp2 = jnp.pad(h2 * (2 * R) + y2, pad2, constant_values=N * 2 * R)
    idxq2 = jnp.pad(t2 * (2 * R) + R + y2, pad2, constant_values=N * 2 * R)
    hpad2 = jnp.pad(h2, pad2, constant_values=N).reshape(NW, NCH2, CH)
    idxp2 = idxp2.reshape(NW, NCH2, CH)
    idxq2 = idxq2.reshape(NW, NCH2, CH)

    pad4 = ((0, 0), (0, EW4_PAD - EW4))
    hpad4 = jnp.pad(head.reshape(NS, EW4), pad4, constant_values=N)
    t2pad4 = jnp.pad(tail.reshape(NS, EW4) * NC, pad4, constant_values=NC * N)

    res = entity_emb
    cur = entity_emb
    for _ in range(N_HOPS):
        pq = _pq_call(cur, relation_emb, W)
        pqpad = jnp.pad(pq.reshape(-1), (0, 2 * R))
        e, s1p = _edge_pass1(pqpad, idxp2, idxq2, hpad2)
        s1p = s1p.reshape(NW, N)
        mhat = jnp.pad(_mhat_call(s1p).reshape(-1), (0, 16))
        ebits = lax.bitcast_convert_type(
            jnp.pad(e.reshape(NS, EW4), pad4, constant_values=-1e30), jnp.int32)
        het = jnp.stack([hpad4, t2pad4, ebits], axis=2)
        het = het.reshape(NS, NCH4, CH4, 3).transpose(0, 1, 3, 2)
        emb2 = jnp.pad(cur.reshape(NC * N, H), ((0, NC), (0, 0)))
        u_halves, s2p = _edge_pass2(emb2, het, mhat)
        s2p = s2p.reshape(NW, N)
        cur, res = _finish_call(
            u_halves[:N2], u_halves[N2:], jnp.transpose(s2p), cur, res)
    return res
